# baseline (speedup 1.0000x reference)
import jax
import jax.numpy as jnp
from jax import lax
from jax.experimental import pallas as pl
from jax.experimental.pallas import tpu as pltpu

ROWS = 4096
FIELD_NUM = 26
EMBED_DIM = 64
NCAND = 128
BLOCK = 256


def _body(cand_ref, x_ref, o_ref, mask_ref):
    @pl.when(pl.program_id(0) == 0)
    def _():
        f = lax.broadcasted_iota(jnp.int32, (FIELD_NUM, EMBED_DIM), 0)
        e = lax.broadcasted_iota(jnp.int32, (FIELD_NUM, EMBED_DIM), 1)
        cols = f * EMBED_DIM + e

        def step(j, m):
            return jnp.where(cols == cand_ref[j], 0.0, m)

        mask_ref[...] = lax.fori_loop(
            0, NCAND, step, jnp.ones((FIELD_NUM, EMBED_DIM), jnp.float32)
        )

    o_ref[...] = x_ref[...] * mask_ref[...][None, :, :]


def kernel(embed, cand):
    return pl.pallas_call(
        _body,
        grid=(ROWS // BLOCK,),
        in_specs=[
            pl.BlockSpec(memory_space=pltpu.SMEM),
            pl.BlockSpec((BLOCK, FIELD_NUM, EMBED_DIM), lambda i: (i, 0, 0)),
        ],
        out_specs=pl.BlockSpec((BLOCK, FIELD_NUM, EMBED_DIM), lambda i: (i, 0, 0)),
        out_shape=jax.ShapeDtypeStruct((ROWS, FIELD_NUM, EMBED_DIM), jnp.float32),
        scratch_shapes=[pltpu.VMEM((FIELD_NUM, EMBED_DIM), jnp.float32)],
    )(cand, embed)


# speedup vs baseline: 7.1320x; 7.1320x over previous
import jax
import jax.numpy as jnp
from jax import lax
from jax.experimental import pallas as pl
from jax.experimental.pallas import tpu as pltpu

ROWS = 4096
FIELD_NUM = 26
EMBED_DIM = 64
COLS = FIELD_NUM * EMBED_DIM
NCAND = 128
BC = 512


def _body(cand_ref, x_ref, o_ref):
    o_ref[...] = x_ref[...]

    def zero_row(j, carry):
        o_ref[pl.ds(cand_ref[j], 1), :] = jnp.zeros((1, BC), jnp.float32)
        return carry

    lax.fori_loop(0, NCAND, zero_row, 0)


def kernel(embed, cand):
    xt = embed.transpose(1, 2, 0).reshape(COLS, ROWS)
    out = pl.pallas_call(
        _body,
        grid=(ROWS // BC,),
        in_specs=[
            pl.BlockSpec(memory_space=pltpu.SMEM),
            pl.BlockSpec((COLS, BC), lambda i: (0, i)),
        ],
        out_specs=pl.BlockSpec((COLS, BC), lambda i: (0, i)),
        out_shape=jax.ShapeDtypeStruct((COLS, ROWS), jnp.float32),
    )(cand, xt)
    return out.reshape(FIELD_NUM, EMBED_DIM, ROWS).transpose(2, 0, 1)
